# baseline (speedup 1.0000x reference)
import jax
import jax.numpy as jnp
from jax import lax
from jax.experimental import pallas as pl
from jax.experimental.pallas import tpu as pltpu
from jax.experimental.pallas import tpu_sc as plsc

SEQ = 4096
BATCH = 4
D_MODEL = 1024

NC = 2
NS = 16
NW = NC * NS
ROWS_PW = SEQ // NW
T = 4
CHUNKS = ROWS_PW // T
LANES = 16
DGRP = D_MODEL // LANES
NIN = 3
NOUT = 2


def _sc_body(x_hbm, t_hbm, o_hbm, xb, tb, ob, x_sem, t_sem, o_sem):
    c = lax.axis_index("c")
    s = lax.axis_index("s")
    wid = s * NC + c
    base = wid * ROWS_PW

    def start_in(slot, chunk):
        t0 = base + chunk * T
        dx = pltpu.async_copy(x_hbm.at[pl.ds(t0, T)], xb.at[slot],
                              x_sem.at[slot])
        dt = pltpu.async_copy(t_hbm.at[pl.ds(t0, T)], tb.at[slot],
                              t_sem.at[slot])
        return dx, dt

    def add_chunk(islot, oslot):
        def body(j, carry):
            t = j // DGRP
            d = (j % DGRP) * LANES
            tv = tb[islot, t, pl.ds(d, LANES)]
            for b in range(BATCH):
                ob[oslot, t, b, pl.ds(d, LANES)] = (
                    xb[islot, t, b, pl.ds(d, LANES)] + tv)
            return carry
        lax.fori_loop(0, T * DGRP, body, 0, unroll=8)

    in_flight = {}
    out_flight = {}
    for p in range(min(NIN, CHUNKS)):
        in_flight[p] = start_in(p % NIN, p)
    for k in range(CHUNKS):
        islot = k % NIN
        oslot = k % NOUT
        dx, dt = in_flight.pop(k)
        dx.wait()
        dt.wait()
        if k - NOUT in out_flight:
            out_flight.pop(k - NOUT).wait()
        add_chunk(islot, oslot)
        out_flight[k] = pltpu.async_copy(
            ob.at[oslot], o_hbm.at[pl.ds(base + k * T, T)],
            o_sem.at[oslot])
        nxt = k + NIN
        if nxt < CHUNKS:
            in_flight[nxt] = start_in(nxt % NIN, nxt)
    for k in sorted(out_flight):
        out_flight.pop(k).wait()


@jax.jit
def kernel(x, table):
    s, b, d = x.shape
    mesh = plsc.VectorSubcoreMesh(core_axis_name="c", subcore_axis_name="s")
    f = pl.kernel(
        _sc_body,
        out_type=jax.ShapeDtypeStruct((s, b, d), x.dtype),
        mesh=mesh,
        scratch_types=[
            pltpu.VMEM((NIN, T, BATCH, D_MODEL), jnp.float32),
            pltpu.VMEM((NIN, T, D_MODEL), jnp.float32),
            pltpu.VMEM((NOUT, T, BATCH, D_MODEL), jnp.float32),
            pltpu.SemaphoreType.DMA((NIN,)),
            pltpu.SemaphoreType.DMA((NIN,)),
            pltpu.SemaphoreType.DMA((NOUT,)),
        ],
    )
    return f(x, table)


# speedup vs baseline: 2.1455x; 2.1455x over previous
import jax
import jax.numpy as jnp
from jax.experimental import pallas as pl

_BS = 512


def _pe_kernel(x_ref, t_ref, o_ref):
    o_ref[...] = x_ref[...] + t_ref[...][:, None, :]


@jax.jit
def kernel(x, table):
    s, b, d = x.shape
    return pl.pallas_call(
        _pe_kernel,
        grid=(s // _BS,),
        in_specs=[
            pl.BlockSpec((_BS, b, d), lambda i: (i, 0, 0)),
            pl.BlockSpec((_BS, d), lambda i: (i, 0)),
        ],
        out_specs=pl.BlockSpec((_BS, b, d), lambda i: (i, 0, 0)),
        out_shape=jax.ShapeDtypeStruct((s, b, d), x.dtype),
    )(x, table)
